# Initial kernel scaffold; baseline (speedup 1.0000x reference)
import jax
import jax.numpy as jnp
from jax.experimental import pallas as pl


def kernel(node_index, x, edge_index, W1, b1, g1, beta1, W2, b2, g2, beta2, W3, b3):
    raise NotImplementedError("write your pallas kernel here")


# speedup vs baseline: 10.5618x; 10.5618x over previous
import functools

import jax
import jax.numpy as jnp
from jax import lax
from jax.experimental import pallas as pl
from jax.experimental.pallas import tpu as pltpu
from jax.experimental.pallas import tpu_sc as plsc

N = 10000
D = 128
DOUT = 40
E = 320000
EPS = 1e-5

NC = 2
NS = 16
L = 16

SUB_ROWS = 624
TAIL_BASE = SUB_ROWS * NS
TAIL_ROWS = N - TAIL_BASE
K_E = 80
EDGES_PER_TILE = E // (NC * NS)

_mesh = plsc.VectorSubcoreMesh(core_axis_name="c", subcore_axis_name="s")


def _init_acc(u_hbm, acc_sh, s):
    rs = s * SUB_ROWS
    pltpu.sync_copy(u_hbm.at[pl.ds(rs, SUB_ROWS)],
                    acc_sh.at[pl.ds(rs, SUB_ROWS)])

    @pl.when(s == NS - 1)
    def _():
        pltpu.sync_copy(u_hbm.at[pl.ds(TAIL_BASE, TAIL_ROWS)],
                        acc_sh.at[pl.ds(TAIL_BASE, TAIL_ROWS)])


def _write_acc(acc_sh, out_hbm, s):
    rs = s * SUB_ROWS
    pltpu.sync_copy(acc_sh.at[pl.ds(rs, SUB_ROWS)],
                    out_hbm.at[pl.ds(rs, SUB_ROWS)])

    @pl.when(s == NS - 1)
    def _():
        pltpu.sync_copy(acc_sh.at[pl.ds(TAIL_BASE, TAIL_ROWS)],
                        out_hbm.at[pl.ds(TAIL_BASE, TAIL_ROWS)])


@functools.partial(
    pl.kernel,
    out_type=[
        jax.ShapeDtypeStruct((N, D), jnp.float32),
        jax.ShapeDtypeStruct((N, D), jnp.float32),
    ],
    mesh=_mesh,
    scratch_types=[
        pltpu.VMEM((K_E,), jnp.int32),
        pltpu.VMEM((K_E, D), jnp.float32),
        pltpu.VMEM_SHARED((N, D), jnp.float32),
    ],
)
def _sc_deg(x_hbm, dst_hbm, d0_hbm, d1_hbm, idx_v, ones_v, acc_sh):
    c = lax.axis_index("c")
    s = lax.axis_index("s")

    @pl.loop(0, K_E)
    def _(r):
        @pl.loop(0, D, step=L)
        def _(j):
            ones_v[r, pl.ds(j, L)] = jnp.full((L,), 1.0, jnp.float32)

    _init_acc(x_hbm, acc_sh, s)
    plsc.subcore_barrier()

    base = (c * NS + s) * EDGES_PER_TILE

    @pl.loop(0, EDGES_PER_TILE, step=K_E)
    def _(i):
        pltpu.sync_copy(dst_hbm.at[pl.ds(base + i, K_E)], idx_v)
        pltpu.sync_copy(ones_v, acc_sh.at[idx_v], add=True)

    plsc.subcore_barrier()

    @pl.when(c == 0)
    def _():
        _write_acc(acc_sh, d0_hbm, s)

    @pl.when(c == 1)
    def _():
        _write_acc(acc_sh, d1_hbm, s)


@functools.partial(
    pl.kernel,
    out_type=[
        jax.ShapeDtypeStruct((N, D), jnp.float32),
        jax.ShapeDtypeStruct((N, D), jnp.float32),
    ],
    mesh=_mesh,
    scratch_types=[
        pltpu.VMEM((K_E,), jnp.int32),
        pltpu.VMEM((K_E,), jnp.int32),
        pltpu.VMEM((K_E, D), jnp.float32),
        pltpu.VMEM_SHARED((N, D), jnp.float32),
    ],
)
def _sc_prop(u_hbm, src_hbm, dst_hbm, s0_hbm, s1_hbm,
             srcv, dstv, rows_v, acc_sh):
    c = lax.axis_index("c")
    s = lax.axis_index("s")

    _init_acc(u_hbm, acc_sh, s)
    plsc.subcore_barrier()

    base = (c * NS + s) * EDGES_PER_TILE

    @pl.loop(0, EDGES_PER_TILE, step=K_E)
    def _(i):
        pltpu.sync_copy(src_hbm.at[pl.ds(base + i, K_E)], srcv)
        pltpu.sync_copy(dst_hbm.at[pl.ds(base + i, K_E)], dstv)
        pltpu.sync_copy(u_hbm.at[srcv], rows_v)
        pltpu.sync_copy(rows_v, acc_sh.at[dstv], add=True)

    plsc.subcore_barrier()

    @pl.when(c == 0)
    def _():
        _write_acc(acc_sh, s0_hbm, s)

    @pl.when(c == 1)
    def _():
        _write_acc(acc_sh, s1_hbm, s)


BLK = 2000
GRID = N // BLK


def _tc_l1_body(x_ref, w_ref, d0_ref, d1_ref, u_ref, dinv_ref):
    x = x_ref[...]
    count = (d0_ref[...] - x) + (d1_ref[...] - x)
    dinv = lax.rsqrt(count + 1.0)
    dinv_ref[...] = dinv[:, :16]
    h = jnp.dot(x, w_ref[...], preferred_element_type=jnp.float32)
    u_ref[...] = h * dinv[:, 0:1]


_tc_l1 = pl.pallas_call(
    _tc_l1_body,
    grid=(GRID,),
    in_specs=[
        pl.BlockSpec((BLK, D), lambda i: (i, 0)),
        pl.BlockSpec((D, D), lambda i: (0, 0)),
        pl.BlockSpec((BLK, D), lambda i: (i, 0)),
        pl.BlockSpec((BLK, D), lambda i: (i, 0)),
    ],
    out_specs=[
        pl.BlockSpec((BLK, D), lambda i: (i, 0)),
        pl.BlockSpec((BLK, 16), lambda i: (i, 0)),
    ],
    out_shape=[
        jax.ShapeDtypeStruct((N, D), jnp.float32),
        jax.ShapeDtypeStruct((N, 16), jnp.float32),
    ],
)


def _tc_stats_body(s0_ref, s1_ref, u_ref, dinv_ref, b_ref, y_ref, sums_ref):
    i = pl.program_id(0)
    sfull = (s0_ref[...] - u_ref[...]) + s1_ref[...]
    y = dinv_ref[:, 0:1] * sfull + b_ref[...]
    y_ref[...] = y

    @pl.when(i == 0)
    def _():
        sums_ref[...] = jnp.zeros_like(sums_ref)

    sums_ref[0:1, :] = sums_ref[0:1, :] + jnp.sum(y, axis=0, keepdims=True)
    sums_ref[1:2, :] = sums_ref[1:2, :] + jnp.sum(y * y, axis=0, keepdims=True)


_tc_stats = pl.pallas_call(
    _tc_stats_body,
    grid=(GRID,),
    in_specs=[
        pl.BlockSpec((BLK, D), lambda i: (i, 0)),
        pl.BlockSpec((BLK, D), lambda i: (i, 0)),
        pl.BlockSpec((BLK, D), lambda i: (i, 0)),
        pl.BlockSpec((BLK, 16), lambda i: (i, 0)),
        pl.BlockSpec((1, D), lambda i: (0, 0)),
    ],
    out_specs=[
        pl.BlockSpec((BLK, D), lambda i: (i, 0)),
        pl.BlockSpec((8, D), lambda i: (0, 0)),
    ],
    out_shape=[
        jax.ShapeDtypeStruct((N, D), jnp.float32),
        jax.ShapeDtypeStruct((8, D), jnp.float32),
    ],
)


def _tc_apply_body(y_ref, sums_ref, g_ref, beta_ref, dinv_ref, w_ref,
                   u_ref, *, with_matmul):
    mean = sums_ref[0:1, :] * (1.0 / N)
    ex2 = sums_ref[1:2, :] * (1.0 / N)
    var = ex2 - mean * mean
    xn = (y_ref[...] - mean) * lax.rsqrt(var + EPS)
    z = jnp.maximum(g_ref[...] * xn + beta_ref[...], 0.0)
    if with_matmul:
        u = jnp.dot(z, w_ref[...], preferred_element_type=jnp.float32)
    else:
        u = z
    u_ref[...] = u * dinv_ref[:, 0:1]


def _make_tc_apply(with_matmul):
    return pl.pallas_call(
        functools.partial(_tc_apply_body, with_matmul=with_matmul),
        grid=(GRID,),
        in_specs=[
            pl.BlockSpec((BLK, D), lambda i: (i, 0)),
            pl.BlockSpec((8, D), lambda i: (0, 0)),
            pl.BlockSpec((1, D), lambda i: (0, 0)),
            pl.BlockSpec((1, D), lambda i: (0, 0)),
            pl.BlockSpec((BLK, 16), lambda i: (i, 0)),
            pl.BlockSpec((D, D), lambda i: (0, 0)),
        ],
        out_specs=pl.BlockSpec((BLK, D), lambda i: (i, 0)),
        out_shape=jax.ShapeDtypeStruct((N, D), jnp.float32),
    )


_tc_apply_mm = _make_tc_apply(True)
_tc_apply_id = _make_tc_apply(False)


def _tc_out_body(s0_ref, s1_ref, u_ref, dinv_ref, w_ref, b_ref, o_ref):
    t = dinv_ref[:, 0:1] * ((s0_ref[...] - u_ref[...]) + s1_ref[...])
    logits = jnp.dot(t, w_ref[...], preferred_element_type=jnp.float32) + b_ref[...]
    m = jnp.max(logits, axis=1, keepdims=True)
    xs = logits - m
    lse = jnp.log(jnp.sum(jnp.exp(xs), axis=1, keepdims=True))
    o_ref[...] = xs - lse


_tc_out = pl.pallas_call(
    _tc_out_body,
    grid=(GRID,),
    in_specs=[
        pl.BlockSpec((BLK, D), lambda i: (i, 0)),
        pl.BlockSpec((BLK, D), lambda i: (i, 0)),
        pl.BlockSpec((BLK, D), lambda i: (i, 0)),
        pl.BlockSpec((BLK, 16), lambda i: (i, 0)),
        pl.BlockSpec((D, DOUT), lambda i: (0, 0)),
        pl.BlockSpec((1, DOUT), lambda i: (0, 0)),
    ],
    out_specs=pl.BlockSpec((BLK, DOUT), lambda i: (i, 0)),
    out_shape=jax.ShapeDtypeStruct((N, DOUT), jnp.float32),
)


def kernel(node_index, x, edge_index, W1, b1, g1, beta1, W2, b2, g2, beta2,
           W3, b3):
    del node_index
    src = edge_index[0]
    dst = edge_index[1]

    d0, d1 = _sc_deg(x, dst)
    u1, dinv = _tc_l1(x, W1, d0, d1)
    s0, s1 = _sc_prop(u1, src, dst)
    y1, sums1 = _tc_stats(s0, s1, u1, dinv, b1.reshape(1, D))
    u2 = _tc_apply_mm(y1, sums1, g1.reshape(1, D), beta1.reshape(1, D),
                      dinv, W2)
    t0, t1 = _sc_prop(u2, src, dst)
    y2, sums2 = _tc_stats(t0, t1, u2, dinv, b2.reshape(1, D))
    u3 = _tc_apply_id(y2, sums2, g2.reshape(1, D), beta2.reshape(1, D),
                      dinv, W2)
    r0, r1 = _sc_prop(u3, src, dst)
    return _tc_out(r0, r1, u3, dinv, W3, b3.reshape(1, DOUT))


# speedup vs baseline: 22.2550x; 2.1071x over previous
import functools

import jax
import jax.numpy as jnp
from jax import lax
from jax.experimental import pallas as pl
from jax.experimental.pallas import tpu as pltpu
from jax.experimental.pallas import tpu_sc as plsc

N = 10000
D = 128
DOUT = 40
E = 320000
EPS = 1e-5

NC = 2
NS = 16
L = 16

SUB_ROWS = 624
TAIL_BASE = SUB_ROWS * NS
TAIL_ROWS = N - TAIL_BASE
CHK = 128
NCHUNK = E // CHK
TILE_CHUNKS = NCHUNK // (NC * NS)
EXTRA_BASE = TILE_CHUNKS * NC * NS

_mesh = plsc.VectorSubcoreMesh(core_axis_name="c", subcore_axis_name="s")


def _init_acc(u_hbm, acc_sh, s):
    rs = s * SUB_ROWS
    pltpu.sync_copy(u_hbm.at[pl.ds(rs, SUB_ROWS)],
                    acc_sh.at[pl.ds(rs, SUB_ROWS)])

    @pl.when(s == NS - 1)
    def _():
        pltpu.sync_copy(u_hbm.at[pl.ds(TAIL_BASE, TAIL_ROWS)],
                        acc_sh.at[pl.ds(TAIL_BASE, TAIL_ROWS)])


def _write_acc(acc_sh, out_hbm, s):
    rs = s * SUB_ROWS
    pltpu.sync_copy(acc_sh.at[pl.ds(rs, SUB_ROWS)],
                    out_hbm.at[pl.ds(rs, SUB_ROWS)])

    @pl.when(s == NS - 1)
    def _():
        pltpu.sync_copy(acc_sh.at[pl.ds(TAIL_BASE, TAIL_ROWS)],
                        out_hbm.at[pl.ds(TAIL_BASE, TAIL_ROWS)])


@functools.partial(
    pl.kernel,
    out_type=[
        jax.ShapeDtypeStruct((N, D), jnp.float32),
        jax.ShapeDtypeStruct((N, D), jnp.float32),
    ],
    mesh=_mesh,
    scratch_types=[
        pltpu.VMEM((4, 2, CHK), jnp.int32),
        pltpu.VMEM((CHK, D), jnp.float32),
        pltpu.VMEM_SHARED((N, D), jnp.float32),
        pltpu.SemaphoreType.DMA,
        pltpu.SemaphoreType.DMA,
        pltpu.SemaphoreType.DMA,
        pltpu.SemaphoreType.DMA,
        pltpu.SemaphoreType.DMA,
        pltpu.SemaphoreType.DMA,
    ],
)
def _sc_deg(x_hbm, edge_hbm, d0_hbm, d1_hbm, ev, ones_v, acc_sh,
            si0, si1, si2, si3, ss0, ss1):
    c = lax.axis_index("c")
    s = lax.axis_index("s")
    wid = c * NS + s
    cbase = wid * TILE_CHUNKS

    @pl.loop(0, CHK)
    def _(r):
        @pl.loop(0, D, step=L)
        def _(j):
            ones_v[r, pl.ds(j, L)] = jnp.full((L,), 1.0, jnp.float32)

    _init_acc(x_hbm, acc_sh, s)
    plsc.subcore_barrier()

    semi = (si0, si1, si2, si3)
    sems = (ss0, ss1)

    def i_start(q, cg):
        pltpu.async_copy(edge_hbm.at[:, pl.ds(cg * CHK, CHK)], ev.at[q],
                         semi[q])

    def i_wait(q):
        pltpu.make_async_copy(edge_hbm.at[:, pl.ds(0, CHK)], ev.at[q],
                              semi[q]).wait()

    def s_start(b, q):
        pltpu.async_copy(ones_v, acc_sh.at[ev.at[q, 1]], sems[b], add=True)

    def s_wait(b):
        pltpu.make_async_copy(ones_v, acc_sh.at[ev.at[0, 1]], sems[b]).wait()

    @pl.when(wid < 4)
    def _():
        pltpu.sync_copy(edge_hbm.at[:, pl.ds((EXTRA_BASE + wid) * CHK, CHK)],
                        ev.at[0])
        pltpu.sync_copy(ones_v, acc_sh.at[ev.at[0, 1]], add=True)

    for q in range(4):
        i_start(q, cbase + q)
    for k in range(2):
        i_wait(k)
        s_start(k, k)

    @pl.loop(2, TILE_CHUNKS, step=4)
    def _(j):
        for r in range(4):
            q = (2 + r) % 4
            qn = r % 4
            b = r % 2
            i_wait(q)
            s_wait(b)
            i_start(qn, cbase + j + r + 2)
            s_start(b, q)

    s_wait(0)
    s_wait(1)
    i_wait(2)
    i_wait(3)

    plsc.subcore_barrier()

    @pl.when(c == 0)
    def _():
        _write_acc(acc_sh, d0_hbm, s)

    @pl.when(c == 1)
    def _():
        _write_acc(acc_sh, d1_hbm, s)


@functools.partial(
    pl.kernel,
    out_type=[
        jax.ShapeDtypeStruct((N, D), jnp.float32),
        jax.ShapeDtypeStruct((N, D), jnp.float32),
    ],
    mesh=_mesh,
    scratch_types=[
        pltpu.VMEM((4, 2, CHK), jnp.int32),
        pltpu.VMEM((2, CHK, D), jnp.float32),
        pltpu.VMEM_SHARED((N, D), jnp.float32),
        pltpu.SemaphoreType.DMA,
        pltpu.SemaphoreType.DMA,
        pltpu.SemaphoreType.DMA,
        pltpu.SemaphoreType.DMA,
        pltpu.SemaphoreType.DMA,
        pltpu.SemaphoreType.DMA,
        pltpu.SemaphoreType.DMA,
        pltpu.SemaphoreType.DMA,
    ],
)
def _sc_prop(u_hbm, edge_hbm, s0_hbm, s1_hbm, ev, rows_v, acc_sh,
             si0, si1, si2, si3, sg0, sg1, ss0, ss1):
    c = lax.axis_index("c")
    s = lax.axis_index("s")
    wid = c * NS + s
    cbase = wid * TILE_CHUNKS

    _init_acc(u_hbm, acc_sh, s)
    plsc.subcore_barrier()

    semi = (si0, si1, si2, si3)
    semg = (sg0, sg1)
    sems = (ss0, ss1)

    def i_start(q, cg):
        pltpu.async_copy(edge_hbm.at[:, pl.ds(cg * CHK, CHK)], ev.at[q],
                         semi[q])

    def i_wait(q):
        pltpu.make_async_copy(edge_hbm.at[:, pl.ds(0, CHK)], ev.at[q],
                              semi[q]).wait()

    def g_start(b, q):
        pltpu.async_copy(u_hbm.at[ev.at[q, 0]], rows_v.at[b], semg[b])

    def g_wait(b):
        pltpu.make_async_copy(u_hbm.at[ev.at[0, 0]], rows_v.at[b],
                              semg[b]).wait()

    def s_start(b, q):
        pltpu.async_copy(rows_v.at[b], acc_sh.at[ev.at[q, 1]], sems[b],
                         add=True)

    def s_wait(b):
        pltpu.make_async_copy(rows_v.at[b], acc_sh.at[ev.at[0, 1]],
                              sems[b]).wait()

    @pl.when(wid < 4)
    def _():
        pltpu.sync_copy(edge_hbm.at[:, pl.ds((EXTRA_BASE + wid) * CHK, CHK)],
                        ev.at[0])
        pltpu.sync_copy(u_hbm.at[ev.at[0, 0]], rows_v.at[0])
        pltpu.sync_copy(rows_v.at[0], acc_sh.at[ev.at[0, 1]], add=True)

    for q in range(4):
        i_start(q, cbase + q)
    for k in range(2):
        i_wait(k)
        g_start(k, k)
        g_wait(k)
        s_start(k, k)

    @pl.loop(2, TILE_CHUNKS, step=4)
    def _(j):
        for r in range(4):
            q = (2 + r) % 4
            qn = r % 4
            b = r % 2
            i_wait(q)
            s_wait(b)
            i_start(qn, cbase + j + r + 2)
            g_start(b, q)
            g_wait(b)
            s_start(b, q)

    s_wait(0)
    s_wait(1)
    i_wait(2)
    i_wait(3)

    plsc.subcore_barrier()

    @pl.when(c == 0)
    def _():
        _write_acc(acc_sh, s0_hbm, s)

    @pl.when(c == 1)
    def _():
        _write_acc(acc_sh, s1_hbm, s)


BLK = 2000
GRID = N // BLK


def _tc_l1_body(x_ref, w_ref, d0_ref, d1_ref, u_ref, dinv_ref):
    x = x_ref[...]
    count = (d0_ref[...] - x) + (d1_ref[...] - x)
    dinv = lax.rsqrt(count + 1.0)
    dinv_ref[...] = dinv[:, :16]
    h = jnp.dot(x, w_ref[...], preferred_element_type=jnp.float32)
    u_ref[...] = h * dinv[:, 0:1]


_tc_l1 = pl.pallas_call(
    _tc_l1_body,
    grid=(GRID,),
    in_specs=[
        pl.BlockSpec((BLK, D), lambda i: (i, 0)),
        pl.BlockSpec((D, D), lambda i: (0, 0)),
        pl.BlockSpec((BLK, D), lambda i: (i, 0)),
        pl.BlockSpec((BLK, D), lambda i: (i, 0)),
    ],
    out_specs=[
        pl.BlockSpec((BLK, D), lambda i: (i, 0)),
        pl.BlockSpec((BLK, 16), lambda i: (i, 0)),
    ],
    out_shape=[
        jax.ShapeDtypeStruct((N, D), jnp.float32),
        jax.ShapeDtypeStruct((N, 16), jnp.float32),
    ],
)


def _tc_stats_body(s0_ref, s1_ref, u_ref, dinv_ref, b_ref, y_ref, sums_ref):
    i = pl.program_id(0)
    sfull = (s0_ref[...] - u_ref[...]) + s1_ref[...]
    y = dinv_ref[:, 0:1] * sfull + b_ref[...]
    y_ref[...] = y

    @pl.when(i == 0)
    def _():
        sums_ref[...] = jnp.zeros_like(sums_ref)

    sums_ref[0:1, :] = sums_ref[0:1, :] + jnp.sum(y, axis=0, keepdims=True)
    sums_ref[1:2, :] = sums_ref[1:2, :] + jnp.sum(y * y, axis=0, keepdims=True)


_tc_stats = pl.pallas_call(
    _tc_stats_body,
    grid=(GRID,),
    in_specs=[
        pl.BlockSpec((BLK, D), lambda i: (i, 0)),
        pl.BlockSpec((BLK, D), lambda i: (i, 0)),
        pl.BlockSpec((BLK, D), lambda i: (i, 0)),
        pl.BlockSpec((BLK, 16), lambda i: (i, 0)),
        pl.BlockSpec((1, D), lambda i: (0, 0)),
    ],
    out_specs=[
        pl.BlockSpec((BLK, D), lambda i: (i, 0)),
        pl.BlockSpec((8, D), lambda i: (0, 0)),
    ],
    out_shape=[
        jax.ShapeDtypeStruct((N, D), jnp.float32),
        jax.ShapeDtypeStruct((8, D), jnp.float32),
    ],
)


def _tc_apply_body(y_ref, sums_ref, g_ref, beta_ref, dinv_ref, w_ref,
                   u_ref, *, with_matmul):
    mean = sums_ref[0:1, :] * (1.0 / N)
    ex2 = sums_ref[1:2, :] * (1.0 / N)
    var = ex2 - mean * mean
    xn = (y_ref[...] - mean) * lax.rsqrt(var + EPS)
    z = jnp.maximum(g_ref[...] * xn + beta_ref[...], 0.0)
    if with_matmul:
        u = jnp.dot(z, w_ref[...], preferred_element_type=jnp.float32)
    else:
        u = z
    u_ref[...] = u * dinv_ref[:, 0:1]


def _make_tc_apply(with_matmul):
    return pl.pallas_call(
        functools.partial(_tc_apply_body, with_matmul=with_matmul),
        grid=(GRID,),
        in_specs=[
            pl.BlockSpec((BLK, D), lambda i: (i, 0)),
            pl.BlockSpec((8, D), lambda i: (0, 0)),
            pl.BlockSpec((1, D), lambda i: (0, 0)),
            pl.BlockSpec((1, D), lambda i: (0, 0)),
            pl.BlockSpec((BLK, 16), lambda i: (i, 0)),
            pl.BlockSpec((D, D), lambda i: (0, 0)),
        ],
        out_specs=pl.BlockSpec((BLK, D), lambda i: (i, 0)),
        out_shape=jax.ShapeDtypeStruct((N, D), jnp.float32),
    )


_tc_apply_mm = _make_tc_apply(True)
_tc_apply_id = _make_tc_apply(False)


def _tc_out_body(s0_ref, s1_ref, u_ref, dinv_ref, w_ref, b_ref, o_ref):
    t = dinv_ref[:, 0:1] * ((s0_ref[...] - u_ref[...]) + s1_ref[...])
    logits = jnp.dot(t, w_ref[...], preferred_element_type=jnp.float32) + b_ref[...]
    m = jnp.max(logits, axis=1, keepdims=True)
    xs = logits - m
    lse = jnp.log(jnp.sum(jnp.exp(xs), axis=1, keepdims=True))
    o_ref[...] = xs - lse


_tc_out = pl.pallas_call(
    _tc_out_body,
    grid=(GRID,),
    in_specs=[
        pl.BlockSpec((BLK, D), lambda i: (i, 0)),
        pl.BlockSpec((BLK, D), lambda i: (i, 0)),
        pl.BlockSpec((BLK, D), lambda i: (i, 0)),
        pl.BlockSpec((BLK, 16), lambda i: (i, 0)),
        pl.BlockSpec((D, DOUT), lambda i: (0, 0)),
        pl.BlockSpec((1, DOUT), lambda i: (0, 0)),
    ],
    out_specs=pl.BlockSpec((BLK, DOUT), lambda i: (i, 0)),
    out_shape=jax.ShapeDtypeStruct((N, DOUT), jnp.float32),
)


def kernel(node_index, x, edge_index, W1, b1, g1, beta1, W2, b2, g2, beta2,
           W3, b3):
    del node_index

    d0, d1 = _sc_deg(x, edge_index)
    u1, dinv = _tc_l1(x, W1, d0, d1)
    s0, s1 = _sc_prop(u1, edge_index)
    y1, sums1 = _tc_stats(s0, s1, u1, dinv, b1.reshape(1, D))
    u2 = _tc_apply_mm(y1, sums1, g1.reshape(1, D), beta1.reshape(1, D),
                      dinv, W2)
    t0, t1 = _sc_prop(u2, edge_index)
    y2, sums2 = _tc_stats(t0, t1, u2, dinv, b2.reshape(1, D))
    u3 = _tc_apply_id(y2, sums2, g2.reshape(1, D), beta2.reshape(1, D),
                      dinv, W2)
    r0, r1 = _sc_prop(u3, edge_index)
    return _tc_out(r0, r1, u3, dinv, W3, b3.reshape(1, DOUT))


# speedup vs baseline: 22.4264x; 1.0077x over previous
import functools

import jax
import jax.numpy as jnp
from jax import lax
from jax.experimental import pallas as pl
from jax.experimental.pallas import tpu as pltpu
from jax.experimental.pallas import tpu_sc as plsc

N = 10000
D = 128
DOUT = 40
E = 320000
EPS = 1e-5

NC = 2
NS = 16
L = 16

SUB_ROWS = 624
TAIL_BASE = SUB_ROWS * NS
TAIL_ROWS = N - TAIL_BASE
CHK = 64
NCHUNK = E // CHK
TILE_CHUNKS = NCHUNK // (NC * NS)
EXTRA_BASE = TILE_CHUNKS * NC * NS
NROW = 4
NEV = 8

_mesh = plsc.VectorSubcoreMesh(core_axis_name="c", subcore_axis_name="s")


def _init_acc(u_hbm, acc_sh, s):
    rs = s * SUB_ROWS
    pltpu.sync_copy(u_hbm.at[pl.ds(rs, SUB_ROWS)],
                    acc_sh.at[pl.ds(rs, SUB_ROWS)])

    @pl.when(s == NS - 1)
    def _():
        pltpu.sync_copy(u_hbm.at[pl.ds(TAIL_BASE, TAIL_ROWS)],
                        acc_sh.at[pl.ds(TAIL_BASE, TAIL_ROWS)])


def _write_acc(acc_sh, out_hbm, s):
    rs = s * SUB_ROWS
    pltpu.sync_copy(acc_sh.at[pl.ds(rs, SUB_ROWS)],
                    out_hbm.at[pl.ds(rs, SUB_ROWS)])

    @pl.when(s == NS - 1)
    def _():
        pltpu.sync_copy(acc_sh.at[pl.ds(TAIL_BASE, TAIL_ROWS)],
                        out_hbm.at[pl.ds(TAIL_BASE, TAIL_ROWS)])


@functools.partial(
    pl.kernel,
    out_type=[
        jax.ShapeDtypeStruct((N, D), jnp.float32),
        jax.ShapeDtypeStruct((N, D), jnp.float32),
    ],
    mesh=_mesh,
    scratch_types=[
        pltpu.VMEM((NEV, 2, CHK), jnp.int32),
        pltpu.VMEM((CHK, D), jnp.float32),
        pltpu.VMEM_SHARED((N, D), jnp.float32),
        pltpu.SemaphoreType.DMA((NEV,)),
        pltpu.SemaphoreType.DMA((NROW,)),
    ],
)
def _sc_deg(x_hbm, edge_hbm, d0_hbm, d1_hbm, ev, ones_v, acc_sh,
            semi, sems):
    c = lax.axis_index("c")
    s = lax.axis_index("s")
    wid = c * NS + s
    cbase = wid * TILE_CHUNKS

    @pl.loop(0, CHK)
    def _(r):
        @pl.loop(0, D, step=L)
        def _(j):
            ones_v[r, pl.ds(j, L)] = jnp.full((L,), 1.0, jnp.float32)

    _init_acc(x_hbm, acc_sh, s)
    plsc.subcore_barrier()

    def i_start(q, cg):
        pltpu.async_copy(edge_hbm.at[cg], ev.at[q], semi.at[q])

    def i_wait(q):
        pltpu.make_async_copy(edge_hbm.at[0], ev.at[q], semi.at[q]).wait()

    def s_start(b, q):
        pltpu.async_copy(ones_v, acc_sh.at[ev.at[q, 1]], sems.at[b],
                         add=True)

    def s_wait(b):
        pltpu.make_async_copy(ones_v, acc_sh.at[ev.at[0, 1]],
                              sems.at[b]).wait()

    @pl.when(wid < 8)
    def _():
        pltpu.sync_copy(edge_hbm.at[EXTRA_BASE + wid], ev.at[0])
        pltpu.sync_copy(ones_v, acc_sh.at[ev.at[0, 1]], add=True)

    def emit(kc, r, with_swait, with_inext):
        q = r % NEV
        b = r % NROW
        qn = (r + 4) % NEV
        i_wait(q)
        if with_swait:
            s_wait(b)
        if with_inext:
            i_start(qn, cbase + kc + 4)
        s_start(b, q)

    for q in range(4):
        i_start(q, cbase + q)
    for k in range(4):
        emit(k, k, False, True)
    for k in range(4, 8):
        emit(k, k, True, True)

    @pl.loop(8, TILE_CHUNKS - 4, step=8)
    def _(j):
        for r in range(8):
            emit(j + r, r, True, True)

    for k in range(TILE_CHUNKS - 4, TILE_CHUNKS):
        emit(k, k % 8, True, False)

    for b in range(NROW):
        s_wait(b)

    plsc.subcore_barrier()

    @pl.when(c == 0)
    def _():
        _write_acc(acc_sh, d0_hbm, s)

    @pl.when(c == 1)
    def _():
        _write_acc(acc_sh, d1_hbm, s)


@functools.partial(
    pl.kernel,
    out_type=[
        jax.ShapeDtypeStruct((N, D), jnp.float32),
        jax.ShapeDtypeStruct((N, D), jnp.float32),
    ],
    mesh=_mesh,
    scratch_types=[
        pltpu.VMEM((NEV, 2, CHK), jnp.int32),
        pltpu.VMEM((NROW, CHK, D), jnp.float32),
        pltpu.VMEM_SHARED((N, D), jnp.float32),
        pltpu.SemaphoreType.DMA((NEV,)),
        pltpu.SemaphoreType.DMA((NROW,)),
        pltpu.SemaphoreType.DMA((NROW,)),
    ],
)
def _sc_prop(u_hbm, edge_hbm, s0_hbm, s1_hbm, ev, rows_v, acc_sh,
             semi, semg, sems):
    c = lax.axis_index("c")
    s = lax.axis_index("s")
    wid = c * NS + s
    cbase = wid * TILE_CHUNKS

    _init_acc(u_hbm, acc_sh, s)
    plsc.subcore_barrier()

    def i_start(q, cg):
        pltpu.async_copy(edge_hbm.at[cg], ev.at[q], semi.at[q])

    def i_wait(q):
        pltpu.make_async_copy(edge_hbm.at[0], ev.at[q], semi.at[q]).wait()

    def g_start(b, q):
        pltpu.async_copy(u_hbm.at[ev.at[q, 0]], rows_v.at[b], semg.at[b])

    def g_wait(b):
        pltpu.make_async_copy(u_hbm.at[ev.at[0, 0]], rows_v.at[b],
                              semg.at[b]).wait()

    def s_start(b, q):
        pltpu.async_copy(rows_v.at[b], acc_sh.at[ev.at[q, 1]], sems.at[b],
                         add=True)

    def s_wait(b):
        pltpu.make_async_copy(rows_v.at[b], acc_sh.at[ev.at[0, 1]],
                              sems.at[b]).wait()

    @pl.when(wid < 8)
    def _():
        pltpu.sync_copy(edge_hbm.at[EXTRA_BASE + wid], ev.at[0])
        pltpu.sync_copy(u_hbm.at[ev.at[0, 0]], rows_v.at[0])
        pltpu.sync_copy(rows_v.at[0], acc_sh.at[ev.at[0, 1]], add=True)

    def emit(kc, r, with_swait, with_next, with_inext):
        q = r % NEV
        q1 = (r + 1) % NEV
        b = r % NROW
        b1 = (r + 1) % NROW
        qn = (r + 5) % NEV
        if with_next:
            i_wait(q1)
            if with_swait:
                s_wait(b1)
            g_start(b1, q1)
            if with_inext:
                i_start(qn, cbase + kc + 5)
        g_wait(b)
        s_start(b, q)

    for q in range(5):
        i_start(q, cbase + q)
    i_wait(0)
    g_start(0, 0)
    for k in range(3):
        emit(k, k, False, True, True)
    for k in range(3, 8):
        emit(k, k, True, True, True)

    @pl.loop(8, TILE_CHUNKS - 4, step=8)
    def _(j):
        for r in range(8):
            emit(j + r, r, True, True, True)

    for k in range(TILE_CHUNKS - 4, TILE_CHUNKS - 1):
        emit(k, k % 8, True, True, False)
    emit(TILE_CHUNKS - 1, (TILE_CHUNKS - 1) % 8, False, False, False)

    for b in range(NROW):
        s_wait(b)
    i_wait(TILE_CHUNKS % NEV)

    plsc.subcore_barrier()

    @pl.when(c == 0)
    def _():
        _write_acc(acc_sh, s0_hbm, s)

    @pl.when(c == 1)
    def _():
        _write_acc(acc_sh, s1_hbm, s)


BLK = 2000
GRID = N // BLK


def _tc_l1_body(x_ref, w_ref, d0_ref, d1_ref, u_ref, dinv_ref):
    x = x_ref[...]
    count = (d0_ref[...] - x) + (d1_ref[...] - x)
    dinv = lax.rsqrt(count + 1.0)
    dinv_ref[...] = dinv[:, :16]
    h = jnp.dot(x, w_ref[...], preferred_element_type=jnp.float32)
    u_ref[...] = h * dinv[:, 0:1]


_tc_l1 = pl.pallas_call(
    _tc_l1_body,
    grid=(GRID,),
    in_specs=[
        pl.BlockSpec((BLK, D), lambda i: (i, 0)),
        pl.BlockSpec((D, D), lambda i: (0, 0)),
        pl.BlockSpec((BLK, D), lambda i: (i, 0)),
        pl.BlockSpec((BLK, D), lambda i: (i, 0)),
    ],
    out_specs=[
        pl.BlockSpec((BLK, D), lambda i: (i, 0)),
        pl.BlockSpec((BLK, 16), lambda i: (i, 0)),
    ],
    out_shape=[
        jax.ShapeDtypeStruct((N, D), jnp.float32),
        jax.ShapeDtypeStruct((N, 16), jnp.float32),
    ],
)


def _tc_stats_body(s0_ref, s1_ref, u_ref, dinv_ref, b_ref, y_ref, sums_ref):
    i = pl.program_id(0)
    sfull = (s0_ref[...] - u_ref[...]) + s1_ref[...]
    y = dinv_ref[:, 0:1] * sfull + b_ref[...]
    y_ref[...] = y

    @pl.when(i == 0)
    def _():
        sums_ref[...] = jnp.zeros_like(sums_ref)

    sums_ref[0:1, :] = sums_ref[0:1, :] + jnp.sum(y, axis=0, keepdims=True)
    sums_ref[1:2, :] = sums_ref[1:2, :] + jnp.sum(y * y, axis=0, keepdims=True)


_tc_stats = pl.pallas_call(
    _tc_stats_body,
    grid=(GRID,),
    in_specs=[
        pl.BlockSpec((BLK, D), lambda i: (i, 0)),
        pl.BlockSpec((BLK, D), lambda i: (i, 0)),
        pl.BlockSpec((BLK, D), lambda i: (i, 0)),
        pl.BlockSpec((BLK, 16), lambda i: (i, 0)),
        pl.BlockSpec((1, D), lambda i: (0, 0)),
    ],
    out_specs=[
        pl.BlockSpec((BLK, D), lambda i: (i, 0)),
        pl.BlockSpec((8, D), lambda i: (0, 0)),
    ],
    out_shape=[
        jax.ShapeDtypeStruct((N, D), jnp.float32),
        jax.ShapeDtypeStruct((8, D), jnp.float32),
    ],
)


def _tc_apply_body(y_ref, sums_ref, g_ref, beta_ref, dinv_ref, w_ref,
                   u_ref, *, with_matmul):
    mean = sums_ref[0:1, :] * (1.0 / N)
    ex2 = sums_ref[1:2, :] * (1.0 / N)
    var = ex2 - mean * mean
    xn = (y_ref[...] - mean) * lax.rsqrt(var + EPS)
    z = jnp.maximum(g_ref[...] * xn + beta_ref[...], 0.0)
    if with_matmul:
        u = jnp.dot(z, w_ref[...], preferred_element_type=jnp.float32)
    else:
        u = z
    u_ref[...] = u * dinv_ref[:, 0:1]


def _make_tc_apply(with_matmul):
    return pl.pallas_call(
        functools.partial(_tc_apply_body, with_matmul=with_matmul),
        grid=(GRID,),
        in_specs=[
            pl.BlockSpec((BLK, D), lambda i: (i, 0)),
            pl.BlockSpec((8, D), lambda i: (0, 0)),
            pl.BlockSpec((1, D), lambda i: (0, 0)),
            pl.BlockSpec((1, D), lambda i: (0, 0)),
            pl.BlockSpec((BLK, 16), lambda i: (i, 0)),
            pl.BlockSpec((D, D), lambda i: (0, 0)),
        ],
        out_specs=pl.BlockSpec((BLK, D), lambda i: (i, 0)),
        out_shape=jax.ShapeDtypeStruct((N, D), jnp.float32),
    )


_tc_apply_mm = _make_tc_apply(True)
_tc_apply_id = _make_tc_apply(False)


def _tc_out_body(s0_ref, s1_ref, u_ref, dinv_ref, w_ref, b_ref, o_ref):
    t = dinv_ref[:, 0:1] * ((s0_ref[...] - u_ref[...]) + s1_ref[...])
    logits = jnp.dot(t, w_ref[...], preferred_element_type=jnp.float32) + b_ref[...]
    m = jnp.max(logits, axis=1, keepdims=True)
    xs = logits - m
    lse = jnp.log(jnp.sum(jnp.exp(xs), axis=1, keepdims=True))
    o_ref[...] = xs - lse


_tc_out = pl.pallas_call(
    _tc_out_body,
    grid=(GRID,),
    in_specs=[
        pl.BlockSpec((BLK, D), lambda i: (i, 0)),
        pl.BlockSpec((BLK, D), lambda i: (i, 0)),
        pl.BlockSpec((BLK, D), lambda i: (i, 0)),
        pl.BlockSpec((BLK, 16), lambda i: (i, 0)),
        pl.BlockSpec((D, DOUT), lambda i: (0, 0)),
        pl.BlockSpec((1, DOUT), lambda i: (0, 0)),
    ],
    out_specs=pl.BlockSpec((BLK, DOUT), lambda i: (i, 0)),
    out_shape=jax.ShapeDtypeStruct((N, DOUT), jnp.float32),
)


def kernel(node_index, x, edge_index, W1, b1, g1, beta1, W2, b2, g2, beta2,
           W3, b3):
    del node_index
    edges = edge_index.reshape(2, NCHUNK, CHK).transpose(1, 0, 2)

    d0, d1 = _sc_deg(x, edges)
    u1, dinv = _tc_l1(x, W1, d0, d1)
    s0, s1 = _sc_prop(u1, edges)
    y1, sums1 = _tc_stats(s0, s1, u1, dinv, b1.reshape(1, D))
    u2 = _tc_apply_mm(y1, sums1, g1.reshape(1, D), beta1.reshape(1, D),
                      dinv, W2)
    t0, t1 = _sc_prop(u2, edges)
    y2, sums2 = _tc_stats(t0, t1, u2, dinv, b2.reshape(1, D))
    u3 = _tc_apply_id(y2, sums2, g2.reshape(1, D), beta2.reshape(1, D),
                      dinv, W2)
    r0, r1 = _sc_prop(u3, edges)
    return _tc_out(r0, r1, u3, dinv, W3, b3.reshape(1, DOUT))
